# baseline (speedup 1.0000x reference)
import jax
import jax.numpy as jnp
from jax.experimental import pallas as pl
from jax.experimental.pallas import tpu as pltpu

DIM = 512
PS = 16
IMG = 224
NH = 2
DH = 64
DEPTH = 6
E = 6
NC = 1000
HP = IMG // PS
NP = HP * HP
PD = PS * PS * 3
SEQ = NP + 1
NPAD = 208
ROWS = 4 * NPAD
B = 4

_DT = jnp.bfloat16
_INTERPRET = False


def _mm(a, b):
    return jax.lax.dot(a.astype(_DT), b.astype(_DT),
                       preferred_element_type=jnp.float32)


def _mm_t(a, b):
    return jax.lax.dot_general(a.astype(_DT), b.astype(_DT),
                               (((1,), (1,)), ((), ())),
                               preferred_element_type=jnp.float32)


def _ln(v, g, b):
    m = jnp.mean(v, axis=-1, keepdims=True)
    c = v - m
    var = jnp.mean(c * c, axis=-1, keepdims=True)
    return c / jnp.sqrt(var + 1e-5) * g + b


def _top1_coefs(logits, val):
    mx = jnp.max(logits, axis=-1, keepdims=True)
    lane = jax.lax.broadcasted_iota(jnp.int32, logits.shape, 1)
    first = jnp.min(jnp.where(logits == mx, lane, logits.shape[-1]),
                    axis=-1, keepdims=True)
    v = val(mx)
    return [jnp.where(first == e, v, 0.0) for e in range(logits.shape[-1])]


def _body(pp_ref, cls_ref, pos_ref,
          pw_ref, pbias_ref, pg1_ref, pb1_ref, pg2_ref, pb2_ref,
          ln1g_ref, ln1b_ref, ln2g_ref, ln2b_ref,
          wqk_ref, bqk_ref, wg_ref, wv_ref, wo_ref,
          rw_ref, rb_ref, ew_ref, eb_ref,
          ng_ref, nb_ref, hw_ref, hb_ref,
          out_ref, x_scr):
    l = pl.program_id(0)

    @pl.when(l == 0)
    def _patch():
        pe = _ln(pp_ref[...], pg1_ref[...], pb1_ref[...])
        emb = _mm(pe, pw_ref[...]) + pbias_ref[...]
        emb = _ln(emb, pg2_ref[...], pb2_ref[...])
        pos = pos_ref[...]
        for bi in range(B):
            base = bi * NPAD
            x_scr[base:base + NPAD, :] = emb[base:base + NPAD] + pos
            x_scr[base:base + 1, :] = cls_ref[...] + pos[0:1]
            x_scr[base + SEQ:base + NPAD, :] = jnp.zeros((NPAD - SEQ, DIM),
                                                         jnp.float32)

    x = x_scr[...]

    xn = _ln(x, ln1g_ref[0], ln1b_ref[0])

    qk = _mm(xn, wqk_ref[0]) + bqk_ref[0]
    qs = [qk[:, h * DH:(h + 1) * DH] * jnp.float32(0.125) for h in range(NH)]
    ks = [qk[:, 128 + h * DH:128 + (h + 1) * DH] for h in range(NH)]

    gl = _mm(xn, wg_ref[0])
    wvm = jnp.concatenate(
        [wv_ref[0, h, e] for h in range(NH) for e in range(E)], axis=1)
    vall = _mm(xn, wvm)
    vs = []
    ocoefs = []
    for h in range(NH):
        vcoefs = _top1_coefs(gl[:, h * E:(h + 1) * E], jax.nn.sigmoid)
        vh = jnp.zeros((ROWS, DH), jnp.float32)
        for e in range(E):
            off = (h * E + e) * DH
            vh = vh + vcoefs[e] * vall[:, off:off + DH]
        vs.append(vh)
        ocoefs.append(_top1_coefs(gl[:, 2 * E + h * E:2 * E + (h + 1) * E],
                                  jax.nn.sigmoid))

    col = jax.lax.broadcasted_iota(jnp.int32, (1, NPAD), 1)
    mbias = jnp.where(col < SEQ, 0.0, -1e30)
    ohs = []
    for h in range(NH):
        parts = []
        for bi in range(B):
            sl = slice(bi * NPAD, (bi + 1) * NPAD)
            att = _mm_t(qs[h][sl], ks[h][sl]) + mbias
            ex = jnp.exp(att)
            rs = 1.0 / jnp.sum(ex, axis=-1, keepdims=True)
            parts.append(_mm(ex, vs[h][sl]) * rs)
        ohs.append(jnp.concatenate(parts, axis=0))

    z = jnp.concatenate(
        [ocoefs[h][e] * ohs[h] for h in range(NH) for e in range(E)], axis=1)
    attn_out = _mm(z, wo_ref[0])

    x = x + attn_out

    xn2 = _ln(x, ln2g_ref[0], ln2b_ref[0])
    rl = _mm(xn2, rw_ref[0]) + rb_ref[0]
    fcoefs = _top1_coefs(
        rl, lambda mx: 1.0 / jnp.sum(jnp.exp(rl - mx), axis=-1, keepdims=True))
    y = _mm(jnp.concatenate(fcoefs, axis=1), eb_ref[0])
    for e in range(E):
        y = y + fcoefs[e] * _mm(xn2, ew_ref[0, e])
    x = x + y

    x_scr[...] = x

    @pl.when(l == DEPTH - 1)
    def _head():
        rows = jnp.concatenate(
            [x[bi * NPAD:bi * NPAD + 1] for bi in range(B)], axis=0)
        hn = _ln(rows, ng_ref[...], nb_ref[...])
        out_ref[...] = _mm(hn, hw_ref[...]) + hb_ref[...]


def kernel(x, params):
    p = params
    L = DEPTH

    patches = x.reshape(B, 3, HP, PS, HP, PS).transpose(0, 2, 4, 3, 5, 1)
    patches = patches.reshape(B, NP, PD)
    pp = jnp.pad(patches, ((0, 0), (1, NPAD - 1 - NP), (0, 0)))
    pp = pp.reshape(ROWS, PD)

    pos = jnp.pad(p['pos_enc'][0], ((0, NPAD - SEQ), (0, 0)))
    cls = p['class_token'].reshape(1, DIM)

    wqk = jnp.concatenate([p['Wq'], p['Wk']], axis=2).astype(_DT)
    bqk = jnp.concatenate([p['bq'], p['bk']], axis=1).reshape(L, 1, 2 * NH * DH)
    wg = jnp.concatenate([p['Wsv'], p['Wso']], axis=2).astype(_DT)
    wv = p['Wv'].astype(_DT)
    wo = p['Wo'].reshape(L, NH * E * DH, DIM).astype(_DT)
    ew = p['expert_W'].astype(_DT)
    hw = p['head_W'].astype(_DT)
    pw = p['patch_W'].astype(_DT)

    inputs = [
        pp, cls, pos,
        pw, p['patch_b'].reshape(1, DIM),
        p['patch_ln1_g'].reshape(1, PD), p['patch_ln1_b'].reshape(1, PD),
        p['patch_ln2_g'].reshape(1, DIM), p['patch_ln2_b'].reshape(1, DIM),
        p['ln1_g'].reshape(L, 1, DIM), p['ln1_b'].reshape(L, 1, DIM),
        p['ln2_g'].reshape(L, 1, DIM), p['ln2_b'].reshape(L, 1, DIM),
        wqk, bqk, wg, wv, wo,
        p['router_W'], p['router_b'].reshape(L, 1, E),
        ew, p['expert_b'],
        p['norm_g'].reshape(1, DIM), p['norm_b'].reshape(1, DIM),
        hw, p['head_b'].reshape(1, NC),
    ]

    def stacked(a):
        shp = a.shape
        return pl.BlockSpec((1,) + shp[1:],
                            lambda l, n=len(shp): (l,) + (0,) * (n - 1))

    def const(a):
        shp = a.shape
        return pl.BlockSpec(shp, lambda l, n=len(shp): (0,) * n)

    per_layer = {9, 10, 11, 12, 13, 14, 15, 16, 17, 18, 19, 20}
    in_specs = [stacked(a) if i in per_layer else const(a)
                for i, a in enumerate(inputs)]

    out = pl.pallas_call(
        _body,
        grid=(DEPTH,),
        in_specs=in_specs,
        out_specs=pl.BlockSpec((B, NC), lambda l: (0, 0)),
        out_shape=jax.ShapeDtypeStruct((B, NC), jnp.float32),
        scratch_shapes=[pltpu.VMEM((ROWS, DIM), jnp.float32)],
        compiler_params=pltpu.CompilerParams(
            dimension_semantics=("arbitrary",)),
        interpret=_INTERPRET,
    )(*inputs)
    return out


# speedup vs baseline: 1.1059x; 1.1059x over previous
import jax
import jax.numpy as jnp
from jax.experimental import pallas as pl
from jax.experimental.pallas import tpu as pltpu

DIM = 512
PS = 16
IMG = 224
NH = 2
DH = 64
DEPTH = 6
E = 6
NC = 1000
HP = IMG // PS
NP = HP * HP
PD = PS * PS * 3
SEQ = NP + 1
NPAD = 208
ROWS = 4 * NPAD
B = 4

_DT = jnp.bfloat16
_INTERPRET = False


def _mm(a, b):
    return jax.lax.dot(a.astype(_DT), b.astype(_DT),
                       preferred_element_type=jnp.float32)


def _mm_t(a, b):
    return jax.lax.dot_general(a.astype(_DT), b.astype(_DT),
                               (((1,), (1,)), ((), ())),
                               preferred_element_type=jnp.float32)


def _ln(v, g, b):
    m = jnp.mean(v, axis=-1, keepdims=True)
    c = v - m
    var = jnp.mean(c * c, axis=-1, keepdims=True)
    return c / jnp.sqrt(var + 1e-5) * g + b


def _top1_coefs(logits, val):
    mx = jnp.max(logits, axis=-1, keepdims=True)
    lane = jax.lax.broadcasted_iota(jnp.int32, logits.shape, 1)
    first = jnp.min(jnp.where(logits == mx, lane, logits.shape[-1]),
                    axis=-1, keepdims=True)
    v = val(mx)
    return [jnp.where(first == e, v, 0.0) for e in range(logits.shape[-1])]


def _body(pp_ref, cls_ref, pos_ref,
          pw_ref, pbias_ref, pg1_ref, pb1_ref, pg2_ref, pb2_ref,
          ln1g_ref, ln1b_ref, ln2g_ref, ln2b_ref,
          wqk_ref, bqk_ref, wg_ref, wv_ref, wo_ref,
          rw_ref, rb_ref, ew_ref, eb_ref,
          ng_ref, nb_ref, hw_ref, hb_ref,
          out_ref, x_scr):
    l = pl.program_id(0)

    @pl.when(l == 0)
    def _patch():
        pe = _ln(pp_ref[...], pg1_ref[...], pb1_ref[...])
        emb = _mm(pe, pw_ref[...]) + pbias_ref[...]
        emb = _ln(emb, pg2_ref[...], pb2_ref[...])
        pos = pos_ref[...]
        for bi in range(B):
            base = bi * NPAD
            x_scr[base:base + NPAD, :] = emb[base:base + NPAD] + pos
            x_scr[base:base + 1, :] = cls_ref[...] + pos[0:1]
            x_scr[base + SEQ:base + NPAD, :] = jnp.zeros((NPAD - SEQ, DIM),
                                                         jnp.float32)

    x = x_scr[...]

    xn = _ln(x, ln1g_ref[0], ln1b_ref[0])

    qk = _mm(xn, wqk_ref[0]) + bqk_ref[0]
    qs = [qk[:, h * DH:(h + 1) * DH] * jnp.float32(0.125) for h in range(NH)]
    ks = [qk[:, 128 + h * DH:128 + (h + 1) * DH] for h in range(NH)]

    gl = _mm(xn, wg_ref[0])
    wvm = jnp.concatenate(
        [wv_ref[0, h, e] for h in range(NH) for e in range(E)], axis=1)
    vall = _mm(xn, wvm)
    vs = []
    ocoefs = []
    for h in range(NH):
        vcoefs = _top1_coefs(gl[:, h * E:(h + 1) * E], jax.nn.sigmoid)
        vh = jnp.zeros((ROWS, DH), jnp.float32)
        for e in range(E):
            off = (h * E + e) * DH
            vh = vh + vcoefs[e] * vall[:, off:off + DH]
        vs.append(vh)
        ocoefs.append(_top1_coefs(gl[:, 2 * E + h * E:2 * E + (h + 1) * E],
                                  jax.nn.sigmoid))

    col = jax.lax.broadcasted_iota(jnp.int32, (1, NPAD), 1)
    mbias = jnp.where(col < SEQ, 0.0, -1e30)
    ohs = []
    for h in range(NH):
        parts = []
        for bi in range(B):
            sl = slice(bi * NPAD, (bi + 1) * NPAD)
            att = _mm_t(qs[h][sl], ks[h][sl]) + mbias
            ex = jnp.exp(att)
            rs = 1.0 / jnp.sum(ex, axis=-1, keepdims=True)
            parts.append(_mm(ex, vs[h][sl]) * rs)
        ohs.append(jnp.concatenate(parts, axis=0))

    z = jnp.concatenate(
        [ocoefs[h][e] * ohs[h] for h in range(NH) for e in range(E)], axis=1)
    attn_out = _mm(z, wo_ref[0])

    x = x + attn_out

    xn2 = _ln(x, ln2g_ref[0], ln2b_ref[0])
    rl = _mm(xn2, rw_ref[0]) + rb_ref[0]
    fcoefs = _top1_coefs(
        rl, lambda mx: 1.0 / jnp.sum(jnp.exp(rl - mx), axis=-1, keepdims=True))
    y = _mm(jnp.concatenate(fcoefs, axis=1), eb_ref[0])
    for e in range(E):
        y = y + fcoefs[e] * _mm(xn2, ew_ref[0, e])
    x = x + y

    x_scr[...] = x

    @pl.when(l == DEPTH - 1)
    def _head():
        rows = jnp.concatenate(
            [x[bi * NPAD:bi * NPAD + 1] for bi in range(B)], axis=0)
        hn = _ln(rows, ng_ref[...], nb_ref[...])
        out_ref[...] = _mm(hn, hw_ref[...]) + hb_ref[...]


def kernel(x, params):
    p = params
    L = DEPTH

    patches = x.reshape(B, 3, HP, PS, HP, PS).transpose(0, 2, 4, 3, 5, 1)
    patches = patches.reshape(B, NP, PD)
    pp = jnp.pad(patches, ((0, 0), (1, NPAD - 1 - NP), (0, 0)))
    pp = pp.reshape(ROWS, PD)

    pos = jnp.pad(p['pos_enc'][0], ((0, NPAD - SEQ), (0, 0)))
    cls = p['class_token'].reshape(1, DIM)

    wqk = jnp.concatenate([p['Wq'], p['Wk']], axis=2)
    bqk = jnp.concatenate([p['bq'], p['bk']], axis=1).reshape(L, 1, 2 * NH * DH)
    wg = jnp.concatenate([p['Wsv'], p['Wso']], axis=2)
    wv = p['Wv']
    wo = p['Wo'].reshape(L, NH * E * DH, DIM)
    ew = p['expert_W']
    hw = p['head_W']
    pw = p['patch_W']

    inputs = [
        pp, cls, pos,
        pw, p['patch_b'].reshape(1, DIM),
        p['patch_ln1_g'].reshape(1, PD), p['patch_ln1_b'].reshape(1, PD),
        p['patch_ln2_g'].reshape(1, DIM), p['patch_ln2_b'].reshape(1, DIM),
        p['ln1_g'].reshape(L, 1, DIM), p['ln1_b'].reshape(L, 1, DIM),
        p['ln2_g'].reshape(L, 1, DIM), p['ln2_b'].reshape(L, 1, DIM),
        wqk, bqk, wg, wv, wo,
        p['router_W'], p['router_b'].reshape(L, 1, E),
        ew, p['expert_b'],
        p['norm_g'].reshape(1, DIM), p['norm_b'].reshape(1, DIM),
        hw, p['head_b'].reshape(1, NC),
    ]

    def stacked(a):
        shp = a.shape
        return pl.BlockSpec((1,) + shp[1:],
                            lambda l, n=len(shp): (l,) + (0,) * (n - 1))

    def const(a):
        shp = a.shape
        return pl.BlockSpec(shp, lambda l, n=len(shp): (0,) * n)

    per_layer = {9, 10, 11, 12, 13, 14, 15, 16, 17, 18, 19, 20}
    in_specs = [stacked(a) if i in per_layer else const(a)
                for i, a in enumerate(inputs)]

    out = pl.pallas_call(
        _body,
        grid=(DEPTH,),
        in_specs=in_specs,
        out_specs=pl.BlockSpec((B, NC), lambda l: (0, 0)),
        out_shape=jax.ShapeDtypeStruct((B, NC), jnp.float32),
        scratch_shapes=[pltpu.VMEM((ROWS, DIM), jnp.float32)],
        compiler_params=pltpu.CompilerParams(
            dimension_semantics=("arbitrary",),
            vmem_limit_bytes=64 * 1024 * 1024),
        interpret=_INTERPRET,
    )(*inputs)
    return out
